# Initial kernel scaffold; baseline (speedup 1.0000x reference)
import jax
import jax.numpy as jnp
from jax.experimental import pallas as pl


def kernel(x, pos, vel, edge_attr, params, edge_index):
    raise NotImplementedError("write your pallas kernel here")


# speedup vs baseline: 8.0866x; 8.0866x over previous
import functools

import numpy as np
import jax
import jax.numpy as jnp
from jax import lax
from jax.experimental import pallas as pl
from jax.experimental.pallas import tpu as pltpu
from jax.experimental.pallas import tpu_sc as plsc

T = 2
N = 10000
E = 320000
F = 128
DE = 16
P = 3
HID = 64
TED = 16
MAX_T = 10000

NC = 2
NS = 16
NW = NC * NS
CH = 128
NCHUNK = E // CH
STR_A = 624
STR_B = N - (NS - 1) * STR_A

BN = 2000
BE = 4000


def _silu(x):
    return x * (1.0 / (1.0 + jnp.exp(-x)))


def _dot(a, b):
    return jnp.dot(a, b, preferred_element_type=jnp.float32)


def _init_body(x_ref, wx_ref, c_ref, out_ref):
    y = _dot(x_ref[...], wx_ref[...])
    out_ref[...] = y[None] + c_ref[0:T][:, None, :]


def _init_call(x, wx, cpad, interpret=False):
    return pl.pallas_call(
        _init_body,
        grid=(N // BN,),
        in_specs=[
            pl.BlockSpec((BN, F), lambda i: (i, 0)),
            pl.BlockSpec((F, F), lambda i: (0, 0)),
            pl.BlockSpec((8, F), lambda i: (0, 0)),
        ],
        out_specs=pl.BlockSpec((T, BN, F), lambda i: (0, i, 0)),
        out_shape=jax.ShapeDtypeStruct((T, N, F), jnp.float32),
        interpret=interpret,
    )(x, wx, cpad)


def _tables_body(xt_ref, pf_ref, wi_ref, wj_ref, b1_ref, td_ref, ts_ref):
    zpad = jnp.zeros((BN, F - HID - 8), jnp.float32)
    for t in range(T):
        xb = xt_ref[t]
        a = _dot(xb, wi_ref[...]) + b1_ref[0:1]
        b = _dot(xb, wj_ref[...])
        pb = pf_ref[t]
        td_ref[t] = jnp.concatenate([a, pb, zpad], axis=1)
        ts_ref[t] = jnp.concatenate([b, pb, zpad], axis=1)


def _tables_call(xt, pf, wi, wj, b1pad, interpret=False):
    return pl.pallas_call(
        _tables_body,
        grid=(N // BN,),
        in_specs=[
            pl.BlockSpec((T, BN, F), lambda i: (0, i, 0)),
            pl.BlockSpec((T, BN, 8), lambda i: (0, i, 0)),
            pl.BlockSpec((F, HID), lambda i: (0, 0)),
            pl.BlockSpec((F, HID), lambda i: (0, 0)),
            pl.BlockSpec((8, HID), lambda i: (0, 0)),
        ],
        out_specs=[
            pl.BlockSpec((T, BN, F), lambda i: (0, i, 0)),
            pl.BlockSpec((T, BN, F), lambda i: (0, i, 0)),
        ],
        out_shape=[
            jax.ShapeDtypeStruct((T, N, F), jnp.float32),
            jax.ShapeDtypeStruct((T, N, F), jnp.float32),
        ],
        interpret=interpret,
    )(xt, pf, wi, wj, b1pad)


def _edge_body(gd_ref, gs_ref, ea_ref, wea_ref, w2_ref, wp1_ref, ec_ref,
               mp_ref):
    ec = ec_ref[...]
    wd = ec[0:1, :HID]
    b2 = ec[1:2, :HID]
    bp1 = ec[2:3, :HID]
    bp2 = ec[3:4, 0:1]
    wp2 = ec[4:5, :HID]
    gd = gd_ref[...]
    gs = gs_ref[...]
    d8 = gd[:, HID:HID + 8] - gs[:, HID:HID + 8]
    d2 = jnp.sum(d8 * d8, axis=1, keepdims=True)
    h0 = gd[:, :HID] + gs[:, :HID] + d2 * wd + _dot(ea_ref[...], wea_ref[...])
    h1 = _silu(h0)
    m = _silu(_dot(h1, w2_ref[...]) + b2)
    q = _silu(_dot(m, wp1_ref[...]) + bp1)
    coef = jnp.sum(q * wp2, axis=1, keepdims=True) + bp2
    zpad = jnp.zeros((BE, F - HID - 8), jnp.float32)
    mp_ref[...] = jnp.concatenate([m, d8 * coef, zpad], axis=1)


def _edge_call(gd, gs, ea, wea, w2, wp1, ec, interpret=False):
    return pl.pallas_call(
        _edge_body,
        grid=(E // BE,),
        in_specs=[
            pl.BlockSpec((BE, F), lambda i: (i, 0)),
            pl.BlockSpec((BE, F), lambda i: (i, 0)),
            pl.BlockSpec((BE, DE), lambda i: (i, 0)),
            pl.BlockSpec((DE, HID), lambda i: (0, 0)),
            pl.BlockSpec((HID, HID), lambda i: (0, 0)),
            pl.BlockSpec((HID, HID), lambda i: (0, 0)),
            pl.BlockSpec((8, 128), lambda i: (0, 0)),
        ],
        out_specs=pl.BlockSpec((BE, F), lambda i: (i, 0)),
        out_shape=jax.ShapeDtypeStruct((E, F), jnp.float32),
        interpret=interpret,
    )(gd, gs, ea, wea, w2, wp1, ec)


def _node_body(xt_ref, om0_ref, om1_ref, pf_ref, vf_ref,
               wv1_ref, wu1x_ref, wu1m_ref, wu2_ref, u_ref, v_ref, nc_ref,
               xo_ref, po_ref, vo_ref):
    ncs = nc_ref[...]
    bv1 = ncs[0:1, :HID]
    bv2 = ncs[1:2, 0:1]
    bu1 = ncs[2:3, :HID]
    bu2 = ncs[3:4, :F]
    wv2 = ncs[4:5, :HID]
    om = (om0_ref, om1_ref)
    xm = []
    for t in range(T):
        x_t = xt_ref[t]
        agg = om[t][0] + om[t][1]
        magg = agg[:, :HID]
        pagg = agg[:, HID:HID + 8]
        hv = _silu(_dot(x_t, wv1_ref[...]) + bv1)
        velc = jnp.sum(hv * wv2, axis=1, keepdims=True) + bv2
        v_new = velc * vf_ref[t] + pagg
        po_ref[t] = pf_ref[t] + v_new
        vo_ref[t] = v_new
        hu = _silu(_dot(x_t, wu1x_ref[...]) + _dot(magg, wu1m_ref[...]) + bu1)
        xm.append(x_t + _dot(hu, wu2_ref[...]) + bu2)
    u_w = u_ref[...]
    v_w = v_ref[...]
    xo_ref[0] = xm[0] + _dot(xm[0], u_w) + _dot(xm[1], v_w)
    xo_ref[1] = xm[1] + _dot(xm[1], u_w) + _dot(xm[0], v_w)


def _node_call(xt, om0, om1, pf, vf, wv1, wu1x, wu1m, wu2, u_w, v_w,
               ncs, interpret=False):
    return pl.pallas_call(
        _node_body,
        grid=(N // BN,),
        in_specs=[
            pl.BlockSpec((T, BN, F), lambda i: (0, i, 0)),
            pl.BlockSpec((NC, BN, F), lambda i: (0, i, 0)),
            pl.BlockSpec((NC, BN, F), lambda i: (0, i, 0)),
            pl.BlockSpec((T, BN, 8), lambda i: (0, i, 0)),
            pl.BlockSpec((T, BN, 8), lambda i: (0, i, 0)),
            pl.BlockSpec((F, HID), lambda i: (0, 0)),
            pl.BlockSpec((F, HID), lambda i: (0, 0)),
            pl.BlockSpec((HID, HID), lambda i: (0, 0)),
            pl.BlockSpec((HID, F), lambda i: (0, 0)),
            pl.BlockSpec((F, F), lambda i: (0, 0)),
            pl.BlockSpec((F, F), lambda i: (0, 0)),
            pl.BlockSpec((8, 128), lambda i: (0, 0)),
        ],
        out_specs=[
            pl.BlockSpec((T, BN, F), lambda i: (0, i, 0)),
            pl.BlockSpec((T, BN, 8), lambda i: (0, i, 0)),
            pl.BlockSpec((T, BN, 8), lambda i: (0, i, 0)),
        ],
        out_shape=[
            jax.ShapeDtypeStruct((T, N, F), jnp.float32),
            jax.ShapeDtypeStruct((T, N, 8), jnp.float32),
            jax.ShapeDtypeStruct((T, N, 8), jnp.float32),
        ],
        interpret=interpret,
    )(xt, om0, om1, pf, vf, wv1, wu1x, wu1m, wu2, u_w, v_w, ncs)


def _sc_mesh():
    return plsc.VectorSubcoreMesh(
        core_axis_name="c", subcore_axis_name="s", num_cores=NC,
        num_subcores=NS)


def _gather_call(tabd, tabs, dst, src):
    @functools.partial(
        pl.kernel,
        out_type=[
            jax.ShapeDtypeStruct((E, F), jnp.float32),
            jax.ShapeDtypeStruct((E, F), jnp.float32),
        ],
        mesh=_sc_mesh(),
        scratch_types=[
            pltpu.VMEM((CH,), jnp.int32),
            pltpu.VMEM((CH,), jnp.int32),
            pltpu.VMEM((CH, F), jnp.float32),
            pltpu.VMEM((CH, F), jnp.float32),
            pltpu.SemaphoreType.DMA,
        ],
    )
    def k(tabd_h, tabs_h, dst_h, src_h, gd_h, gs_h,
          idxd, idxs, bufd, bufs, sem):
        c = lax.axis_index("c")
        s = lax.axis_index("s")
        wid = s * NC + c
        nk = (NCHUNK + NW - 1 - wid) // NW

        def body(kk, carry):
            base = (wid + kk * NW) * CH
            pltpu.sync_copy(dst_h.at[pl.ds(base, CH)], idxd)
            pltpu.sync_copy(src_h.at[pl.ds(base, CH)], idxs)
            d1 = pltpu.async_copy(tabd_h.at[idxd], bufd, sem)
            d2 = pltpu.async_copy(tabs_h.at[idxs], bufs, sem)
            d1.wait()
            d2.wait()
            pltpu.sync_copy(bufd, gd_h.at[pl.ds(base, CH)])
            pltpu.sync_copy(bufs, gs_h.at[pl.ds(base, CH)])
            return carry

        lax.fori_loop(0, nk, body, 0)

    return k(tabd, tabs, dst, src)


def _scatter_call(mp, dst, zm):
    @functools.partial(
        pl.kernel,
        out_type=jax.ShapeDtypeStruct((NC * N, F), jnp.float32),
        mesh=_sc_mesh(),
        scratch_types=[
            pltpu.VMEM((1, CH), jnp.int32),
            pltpu.VMEM((CH, F), jnp.float32),
            pltpu.VMEM_SHARED((N, F), jnp.float32),
        ],
    )
    def k(mp_h, dst_h, zm_h, om_h, idx2, bufm, acc):
        c = lax.axis_index("c")
        s = lax.axis_index("s")
        wid = s * NC + c

        @pl.when(s < NS - 1)
        def _():
            pltpu.sync_copy(zm_h.at[pl.ds(0, STR_A)],
                            acc.at[pl.ds(s * STR_A, STR_A)])

        @pl.when(s == NS - 1)
        def _():
            pltpu.sync_copy(zm_h, acc.at[pl.ds((NS - 1) * STR_A, STR_B)])

        plsc.subcore_barrier()
        nk = (NCHUNK + NW - 1 - wid) // NW

        def body(kk, carry):
            base = (wid + kk * NW) * CH
            pltpu.sync_copy(dst_h.at[pl.ds(base, CH)], idx2.at[0])
            pltpu.sync_copy(mp_h.at[pl.ds(base, CH)], bufm)
            pltpu.sync_copy(bufm, acc.at[idx2.at[0]], add=True)
            return carry

        lax.fori_loop(0, nk, body, 0)
        plsc.subcore_barrier()

        @pl.when(s < NS - 1)
        def _():
            pltpu.sync_copy(acc.at[pl.ds(s * STR_A, STR_A)],
                            om_h.at[pl.ds(c * N + s * STR_A, STR_A)])

        @pl.when(s == NS - 1)
        def _():
            pltpu.sync_copy(acc.at[pl.ds((NS - 1) * STR_A, STR_B)],
                            om_h.at[pl.ds(c * N + (NS - 1) * STR_A, STR_B)])

    return k(mp, dst, zm)


def _pad_rows(v, rows=8):
    v2 = v.reshape(1, -1)
    return jnp.concatenate(
        [v2, jnp.zeros((rows - 1, v2.shape[1]), jnp.float32)], axis=0)


def _pack_consts(rows, width=128):
    out = []
    for r in rows:
        r = jnp.asarray(r, jnp.float32).reshape(-1)
        out.append(jnp.concatenate(
            [r, jnp.zeros((width - r.shape[0],), jnp.float32)]))
    while len(out) < 8:
        out.append(jnp.zeros((width,), jnp.float32))
    return jnp.stack(out)


def kernel(x, pos, vel, edge_attr, params, edge_index):
    src = edge_index[0].astype(jnp.int32)
    dst = edge_index[1].astype(jnp.int32)

    half = TED // 2
    logs = np.log(MAX_T) / (half - 1)
    freqs = np.exp(-np.arange(half) * logs)
    args = np.arange(T)[:, None] * freqs[None, :]
    emb = jnp.asarray(
        np.concatenate([np.sin(args), np.cos(args)], axis=-1), jnp.float32)
    c = emb @ params['lin_w'][F:] + params['lin_b']
    cpad = jnp.concatenate([c, jnp.zeros((8 - T, F), jnp.float32)], axis=0)

    xt = _init_call(x, params['lin_w'][:F], cpad)

    pf = jnp.concatenate([pos, jnp.zeros((N, 8 - P), jnp.float32)], axis=1)
    pf = jnp.broadcast_to(pf[None], (T, N, 8))
    vf = jnp.concatenate([vel, jnp.zeros((N, 8 - P), jnp.float32)], axis=1)
    vf = jnp.broadcast_to(vf[None], (T, N, 8))

    zm = jnp.zeros((STR_B, F), jnp.float32)

    for lp in params['layers']:
        (w1, b1), (w2, b2) = lp['msg']
        w_xi, w_xj, wd, wea = w1[:F], w1[F:2 * F], w1[2 * F], w1[2 * F + 1:]
        (wp1, bp1), (wp2, bp2) = lp['posm']
        (wv1, bv1), (wv2, bv2) = lp['velm']
        (wu1, bu1), (wu2, bu2) = lp['upd']
        wu1x, wu1m = wu1[:F], wu1[F:]
        u_w = (lp['wr'][0] + lp['wr'][1]) * 0.5
        v_w = (lp['wr'][0] - lp['wr'][1]) * 0.5

        tabd, tabs = _tables_call(xt, pf, w_xi, w_xj, _pad_rows(b1))
        ec = _pack_consts([wd, b2, bp1, bp2, wp2.reshape(-1)])
        ncs = _pack_consts([bv1, bv2, bu1, bu2, wv2.reshape(-1)])

        oms = []
        for t in range(T):
            gd, gs = _gather_call(tabd[t], tabs[t], dst, src)
            mp = _edge_call(gd, gs, edge_attr, wea, w2, wp1, ec)
            om = _scatter_call(mp, dst, zm)
            oms.append(om.reshape(NC, N, F))

        xt, pf, vf = _node_call(
            xt, oms[0], oms[1], pf, vf,
            wv1, wu1x, wu1m, wu2, u_w, v_w, ncs)

    return xt, pf[..., :P], vf[..., :P]


# speedup vs baseline: 9.5533x; 1.1814x over previous
import functools

import numpy as np
import jax
import jax.numpy as jnp
from jax import lax
from jax.experimental import pallas as pl
from jax.experimental.pallas import tpu as pltpu
from jax.experimental.pallas import tpu_sc as plsc

T = 2
N = 10000
E = 320000
F = 128
DE = 16
P = 3
HID = 64
TED = 16
MAX_T = 10000

NC = 2
NS = 16
NW = NC * NS
CH = 128
NCHUNK = E // CH
STR_A = 624
STR_B = N - (NS - 1) * STR_A

BN = 2000
BE = 4000


def _silu(x):
    return x * (1.0 / (1.0 + jnp.exp(-x)))


def _dot(a, b):
    return jnp.dot(a, b, preferred_element_type=jnp.float32)


def _init_body(x_ref, wx_ref, c_ref, out_ref):
    y = _dot(x_ref[...], wx_ref[...])
    out_ref[...] = y[None] + c_ref[0:T][:, None, :]


def _init_call(x, wx, cpad, interpret=False):
    return pl.pallas_call(
        _init_body,
        grid=(N // BN,),
        in_specs=[
            pl.BlockSpec((BN, F), lambda i: (i, 0)),
            pl.BlockSpec((F, F), lambda i: (0, 0)),
            pl.BlockSpec((8, F), lambda i: (0, 0)),
        ],
        out_specs=pl.BlockSpec((T, BN, F), lambda i: (0, i, 0)),
        out_shape=jax.ShapeDtypeStruct((T, N, F), jnp.float32),
        interpret=interpret,
    )(x, wx, cpad)


def _tables_body(xt_ref, pf_ref, wi_ref, wj_ref, b1_ref, td_ref, ts_ref):
    zpad = jnp.zeros((BN, F - HID - 8), jnp.float32)
    for t in range(T):
        xb = xt_ref[t]
        a = _dot(xb, wi_ref[...]) + b1_ref[0:1]
        b = _dot(xb, wj_ref[...])
        pb = pf_ref[t]
        td_ref[t] = jnp.concatenate([a, pb, zpad], axis=1)
        ts_ref[t] = jnp.concatenate([b, pb, zpad], axis=1)


def _tables_call(xt, pf, wi, wj, b1pad, interpret=False):
    return pl.pallas_call(
        _tables_body,
        grid=(N // BN,),
        in_specs=[
            pl.BlockSpec((T, BN, F), lambda i: (0, i, 0)),
            pl.BlockSpec((T, BN, 8), lambda i: (0, i, 0)),
            pl.BlockSpec((F, HID), lambda i: (0, 0)),
            pl.BlockSpec((F, HID), lambda i: (0, 0)),
            pl.BlockSpec((8, HID), lambda i: (0, 0)),
        ],
        out_specs=[
            pl.BlockSpec((T, BN, F), lambda i: (0, i, 0)),
            pl.BlockSpec((T, BN, F), lambda i: (0, i, 0)),
        ],
        out_shape=[
            jax.ShapeDtypeStruct((T, N, F), jnp.float32),
            jax.ShapeDtypeStruct((T, N, F), jnp.float32),
        ],
        interpret=interpret,
    )(xt, pf, wi, wj, b1pad)


def _edge_body(gd_ref, gs_ref, ea_ref, wea_ref, w2_ref, wp1_ref, ec_ref,
               mp_ref):
    ec = ec_ref[...]
    wd = ec[0:1, :HID]
    b2 = ec[1:2, :HID]
    bp1 = ec[2:3, :HID]
    bp2 = ec[3:4, 0:1]
    wp2 = ec[4:5, :HID]
    gd = gd_ref[...]
    gs = gs_ref[...]
    d8 = gd[:, HID:HID + 8] - gs[:, HID:HID + 8]
    d2 = jnp.sum(d8 * d8, axis=1, keepdims=True)
    h0 = gd[:, :HID] + gs[:, :HID] + d2 * wd + _dot(ea_ref[...], wea_ref[...])
    h1 = _silu(h0)
    m = _silu(_dot(h1, w2_ref[...]) + b2)
    q = _silu(_dot(m, wp1_ref[...]) + bp1)
    coef = jnp.sum(q * wp2, axis=1, keepdims=True) + bp2
    zpad = jnp.zeros((BE, F - HID - 8), jnp.float32)
    mp_ref[...] = jnp.concatenate([m, d8 * coef, zpad], axis=1)


def _edge_call(gd, gs, ea, wea, w2, wp1, ec, interpret=False):
    return pl.pallas_call(
        _edge_body,
        grid=(E // BE,),
        in_specs=[
            pl.BlockSpec((BE, F), lambda i: (i, 0)),
            pl.BlockSpec((BE, F), lambda i: (i, 0)),
            pl.BlockSpec((BE, DE), lambda i: (i, 0)),
            pl.BlockSpec((DE, HID), lambda i: (0, 0)),
            pl.BlockSpec((HID, HID), lambda i: (0, 0)),
            pl.BlockSpec((HID, HID), lambda i: (0, 0)),
            pl.BlockSpec((8, 128), lambda i: (0, 0)),
        ],
        out_specs=pl.BlockSpec((BE, F), lambda i: (i, 0)),
        out_shape=jax.ShapeDtypeStruct((E, F), jnp.float32),
        interpret=interpret,
    )(gd, gs, ea, wea, w2, wp1, ec)


def _node_body(xt_ref, om0_ref, om1_ref, pf_ref, vf_ref,
               wv1_ref, wu1x_ref, wu1m_ref, wu2_ref, u_ref, v_ref, nc_ref,
               xo_ref, po_ref, vo_ref):
    ncs = nc_ref[...]
    bv1 = ncs[0:1, :HID]
    bv2 = ncs[1:2, 0:1]
    bu1 = ncs[2:3, :HID]
    bu2 = ncs[3:4, :F]
    wv2 = ncs[4:5, :HID]
    om = (om0_ref, om1_ref)
    xm = []
    for t in range(T):
        x_t = xt_ref[t]
        agg = om[t][0] + om[t][1]
        magg = agg[:, :HID]
        pagg = agg[:, HID:HID + 8]
        hv = _silu(_dot(x_t, wv1_ref[...]) + bv1)
        velc = jnp.sum(hv * wv2, axis=1, keepdims=True) + bv2
        v_new = velc * vf_ref[t] + pagg
        po_ref[t] = pf_ref[t] + v_new
        vo_ref[t] = v_new
        hu = _silu(_dot(x_t, wu1x_ref[...]) + _dot(magg, wu1m_ref[...]) + bu1)
        xm.append(x_t + _dot(hu, wu2_ref[...]) + bu2)
    u_w = u_ref[...]
    v_w = v_ref[...]
    xo_ref[0] = xm[0] + _dot(xm[0], u_w) + _dot(xm[1], v_w)
    xo_ref[1] = xm[1] + _dot(xm[1], u_w) + _dot(xm[0], v_w)


def _node_call(xt, om0, om1, pf, vf, wv1, wu1x, wu1m, wu2, u_w, v_w,
               ncs, interpret=False):
    return pl.pallas_call(
        _node_body,
        grid=(N // BN,),
        in_specs=[
            pl.BlockSpec((T, BN, F), lambda i: (0, i, 0)),
            pl.BlockSpec((NC, BN, F), lambda i: (0, i, 0)),
            pl.BlockSpec((NC, BN, F), lambda i: (0, i, 0)),
            pl.BlockSpec((T, BN, 8), lambda i: (0, i, 0)),
            pl.BlockSpec((T, BN, 8), lambda i: (0, i, 0)),
            pl.BlockSpec((F, HID), lambda i: (0, 0)),
            pl.BlockSpec((F, HID), lambda i: (0, 0)),
            pl.BlockSpec((HID, HID), lambda i: (0, 0)),
            pl.BlockSpec((HID, F), lambda i: (0, 0)),
            pl.BlockSpec((F, F), lambda i: (0, 0)),
            pl.BlockSpec((F, F), lambda i: (0, 0)),
            pl.BlockSpec((8, 128), lambda i: (0, 0)),
        ],
        out_specs=[
            pl.BlockSpec((T, BN, F), lambda i: (0, i, 0)),
            pl.BlockSpec((T, BN, 8), lambda i: (0, i, 0)),
            pl.BlockSpec((T, BN, 8), lambda i: (0, i, 0)),
        ],
        out_shape=[
            jax.ShapeDtypeStruct((T, N, F), jnp.float32),
            jax.ShapeDtypeStruct((T, N, 8), jnp.float32),
            jax.ShapeDtypeStruct((T, N, 8), jnp.float32),
        ],
        interpret=interpret,
    )(xt, om0, om1, pf, vf, wv1, wu1x, wu1m, wu2, u_w, v_w, ncs)


def _sc_mesh():
    return plsc.VectorSubcoreMesh(
        core_axis_name="c", subcore_axis_name="s", num_cores=NC,
        num_subcores=NS)


def _gather_call(tabd, tabs, dst, src):
    @functools.partial(
        pl.kernel,
        out_type=[
            jax.ShapeDtypeStruct((E, F), jnp.float32),
            jax.ShapeDtypeStruct((E, F), jnp.float32),
        ],
        mesh=_sc_mesh(),
        scratch_types=[
            pltpu.VMEM((2, CH), jnp.int32),
            pltpu.VMEM((2, CH), jnp.int32),
            pltpu.VMEM((2, CH, F), jnp.float32),
            pltpu.VMEM((2, CH, F), jnp.float32),
            pltpu.SemaphoreType.DMA,
            pltpu.SemaphoreType.DMA,
            pltpu.SemaphoreType.DMA,
        ],
    )
    def k(tabd_h, tabs_h, dst_h, src_h, gd_h, gs_h,
          idxd, idxs, bufd, bufs, semi, semg, semw):
        c = lax.axis_index("c")
        s = lax.axis_index("s")
        wid = s * NC + c
        nk = (NCHUNK + NW - 1 - wid) // NW

        def chunk_base(i):
            return (wid + i * NW) * CH

        def fire_idx(i, slot):
            base = chunk_base(i)
            pltpu.async_copy(dst_h.at[pl.ds(base, CH)], idxd.at[slot], semi)
            pltpu.async_copy(src_h.at[pl.ds(base, CH)], idxs.at[slot], semi)

        def wait_idx(slot):
            pltpu.make_async_copy(dst_h.at[pl.ds(0, CH)], idxd.at[slot],
                                  semi).wait()
            pltpu.make_async_copy(src_h.at[pl.ds(0, CH)], idxs.at[slot],
                                  semi).wait()

        def fire_gather(slot):
            pltpu.async_copy(tabd_h.at[idxd.at[slot]], bufd.at[slot], semg)
            pltpu.async_copy(tabs_h.at[idxs.at[slot]], bufs.at[slot], semg)

        def wait_gather(slot):
            pltpu.make_async_copy(tabd_h.at[idxd.at[slot]], bufd.at[slot],
                                  semg).wait()
            pltpu.make_async_copy(tabs_h.at[idxs.at[slot]], bufs.at[slot],
                                  semg).wait()

        def fire_wb(i, slot):
            base = chunk_base(i)
            pltpu.async_copy(bufd.at[slot], gd_h.at[pl.ds(base, CH)], semw)
            pltpu.async_copy(bufs.at[slot], gs_h.at[pl.ds(base, CH)], semw)

        def wait_wb(slot):
            pltpu.make_async_copy(bufd.at[slot], gd_h.at[pl.ds(0, CH)],
                                  semw).wait()
            pltpu.make_async_copy(bufs.at[slot], gs_h.at[pl.ds(0, CH)],
                                  semw).wait()

        fire_idx(0, 0)

        def body(i, carry):
            slot = lax.rem(i, 2)
            wait_idx(slot)

            @pl.when(i >= 2)
            def _():
                wait_wb(slot)

            fire_gather(slot)

            @pl.when(i >= 1)
            def _():
                wait_gather(1 - slot)
                fire_wb(i - 1, 1 - slot)

            @pl.when(i + 1 < nk)
            def _():
                fire_idx(i + 1, 1 - slot)

            return carry

        lax.fori_loop(0, nk, body, 0)
        last = lax.rem(nk - 1, 2)
        wait_gather(last)
        fire_wb(nk - 1, last)

        @pl.when(nk >= 2)
        def _():
            wait_wb(1 - last)

        wait_wb(last)

    return k(tabd, tabs, dst, src)


def _scatter_call(mp, dst, zm):
    @functools.partial(
        pl.kernel,
        out_type=jax.ShapeDtypeStruct((NC * N, F), jnp.float32),
        mesh=_sc_mesh(),
        scratch_types=[
            pltpu.VMEM((2, CH), jnp.int32),
            pltpu.VMEM((2, CH, F), jnp.float32),
            pltpu.VMEM_SHARED((N, F), jnp.float32),
            pltpu.SemaphoreType.DMA,
            pltpu.SemaphoreType.DMA,
            pltpu.SemaphoreType.DMA,
        ],
    )
    def k(mp_h, dst_h, zm_h, om_h, idx2, bufm, acc, semi, semm, sema):
        c = lax.axis_index("c")
        s = lax.axis_index("s")
        wid = s * NC + c

        @pl.when(s < NS - 1)
        def _():
            pltpu.sync_copy(zm_h.at[pl.ds(0, STR_A)],
                            acc.at[pl.ds(s * STR_A, STR_A)])

        @pl.when(s == NS - 1)
        def _():
            pltpu.sync_copy(zm_h, acc.at[pl.ds((NS - 1) * STR_A, STR_B)])

        plsc.subcore_barrier()
        nk = (NCHUNK + NW - 1 - wid) // NW

        def chunk_base(i):
            return (wid + i * NW) * CH

        def fire_in(i, slot):
            base = chunk_base(i)
            pltpu.async_copy(dst_h.at[pl.ds(base, CH)], idx2.at[slot], semi)
            pltpu.async_copy(mp_h.at[pl.ds(base, CH)], bufm.at[slot], semm)

        def wait_in(slot):
            pltpu.make_async_copy(dst_h.at[pl.ds(0, CH)], idx2.at[slot],
                                  semi).wait()
            pltpu.make_async_copy(mp_h.at[pl.ds(0, CH)], bufm.at[slot],
                                  semm).wait()

        def fire_add(slot):
            pltpu.async_copy(bufm.at[slot], acc.at[idx2.at[slot]], sema,
                             add=True)

        def wait_add(slot):
            pltpu.make_async_copy(bufm.at[slot], acc.at[idx2.at[slot]],
                                  sema).wait()

        fire_in(0, 0)

        def body(i, carry):
            slot = lax.rem(i, 2)
            wait_in(slot)
            fire_add(slot)

            @pl.when(i >= 1)
            def _():
                wait_add(1 - slot)

            @pl.when(i + 1 < nk)
            def _():
                fire_in(i + 1, 1 - slot)

            return carry

        lax.fori_loop(0, nk, body, 0)
        wait_add(lax.rem(nk - 1, 2))
        plsc.subcore_barrier()

        @pl.when(s < NS - 1)
        def _():
            pltpu.sync_copy(acc.at[pl.ds(s * STR_A, STR_A)],
                            om_h.at[pl.ds(c * N + s * STR_A, STR_A)])

        @pl.when(s == NS - 1)
        def _():
            pltpu.sync_copy(acc.at[pl.ds((NS - 1) * STR_A, STR_B)],
                            om_h.at[pl.ds(c * N + (NS - 1) * STR_A, STR_B)])

    return k(mp, dst, zm)


def _pad_rows(v, rows=8):
    v2 = v.reshape(1, -1)
    return jnp.concatenate(
        [v2, jnp.zeros((rows - 1, v2.shape[1]), jnp.float32)], axis=0)


def _pack_consts(rows, width=128):
    out = []
    for r in rows:
        r = jnp.asarray(r, jnp.float32).reshape(-1)
        out.append(jnp.concatenate(
            [r, jnp.zeros((width - r.shape[0],), jnp.float32)]))
    while len(out) < 8:
        out.append(jnp.zeros((width,), jnp.float32))
    return jnp.stack(out)


def kernel(x, pos, vel, edge_attr, params, edge_index):
    src = edge_index[0].astype(jnp.int32)
    dst = edge_index[1].astype(jnp.int32)

    half = TED // 2
    logs = np.log(MAX_T) / (half - 1)
    freqs = np.exp(-np.arange(half) * logs)
    args = np.arange(T)[:, None] * freqs[None, :]
    emb = jnp.asarray(
        np.concatenate([np.sin(args), np.cos(args)], axis=-1), jnp.float32)
    c = emb @ params['lin_w'][F:] + params['lin_b']
    cpad = jnp.concatenate([c, jnp.zeros((8 - T, F), jnp.float32)], axis=0)

    xt = _init_call(x, params['lin_w'][:F], cpad)

    pf = jnp.concatenate([pos, jnp.zeros((N, 8 - P), jnp.float32)], axis=1)
    pf = jnp.broadcast_to(pf[None], (T, N, 8))
    vf = jnp.concatenate([vel, jnp.zeros((N, 8 - P), jnp.float32)], axis=1)
    vf = jnp.broadcast_to(vf[None], (T, N, 8))

    zm = jnp.zeros((STR_B, F), jnp.float32)

    for lp in params['layers']:
        (w1, b1), (w2, b2) = lp['msg']
        w_xi, w_xj, wd, wea = w1[:F], w1[F:2 * F], w1[2 * F], w1[2 * F + 1:]
        (wp1, bp1), (wp2, bp2) = lp['posm']
        (wv1, bv1), (wv2, bv2) = lp['velm']
        (wu1, bu1), (wu2, bu2) = lp['upd']
        wu1x, wu1m = wu1[:F], wu1[F:]
        u_w = (lp['wr'][0] + lp['wr'][1]) * 0.5
        v_w = (lp['wr'][0] - lp['wr'][1]) * 0.5

        tabd, tabs = _tables_call(xt, pf, w_xi, w_xj, _pad_rows(b1))
        ec = _pack_consts([wd, b2, bp1, bp2, wp2.reshape(-1)])
        ncs = _pack_consts([bv1, bv2, bu1, bu2, wv2.reshape(-1)])

        oms = []
        for t in range(T):
            gd, gs = _gather_call(tabd[t], tabs[t], dst, src)
            mp = _edge_call(gd, gs, edge_attr, wea, w2, wp1, ec)
            om = _scatter_call(mp, dst, zm)
            oms.append(om.reshape(NC, N, F))

        xt, pf, vf = _node_call(
            xt, oms[0], oms[1], pf, vf,
            wv1, wu1x, wu1m, wu2, u_w, v_w, ncs)

    return xt, pf[..., :P], vf[..., :P]


# speedup vs baseline: 11.5091x; 1.2047x over previous
import functools

import numpy as np
import jax
import jax.numpy as jnp
from jax import lax
from jax.experimental import pallas as pl
from jax.experimental.pallas import tpu as pltpu
from jax.experimental.pallas import tpu_sc as plsc

T = 2
N = 10000
E = 320000
F = 128
DE = 16
P = 3
HID = 64
TED = 16
MAX_T = 10000

NC = 2
NS = 16
NW = NC * NS
CH = 128
NCHUNK = E // CH
STR_A = 624
STR_B = N - (NS - 1) * STR_A

BN = 2000
BE = 4000


def _silu(x):
    return x * (1.0 / (1.0 + jnp.exp(-x)))


def _dot(a, b):
    return jnp.dot(a, b, preferred_element_type=jnp.float32)


def _init_body(x_ref, wx_ref, c_ref, out_ref):
    y = _dot(x_ref[...], wx_ref[...])
    out_ref[...] = y[None] + c_ref[0:T][:, None, :]


def _init_call(x, wx, cpad, interpret=False):
    return pl.pallas_call(
        _init_body,
        grid=(N // BN,),
        in_specs=[
            pl.BlockSpec((BN, F), lambda i: (i, 0)),
            pl.BlockSpec((F, F), lambda i: (0, 0)),
            pl.BlockSpec((8, F), lambda i: (0, 0)),
        ],
        out_specs=pl.BlockSpec((T, BN, F), lambda i: (0, i, 0)),
        out_shape=jax.ShapeDtypeStruct((T, N, F), jnp.float32),
        interpret=interpret,
    )(x, wx, cpad)


def _tables_body(xt_ref, pf_ref, wi_ref, wj_ref, b1_ref, td_ref, ts_ref):
    zpad = jnp.zeros((BN, F - HID - 8), jnp.float32)
    for t in range(T):
        xb = xt_ref[t]
        a = _dot(xb, wi_ref[...]) + b1_ref[0:1]
        b = _dot(xb, wj_ref[...])
        pb = pf_ref[t]
        td_ref[t] = jnp.concatenate([a, pb, zpad], axis=1)
        ts_ref[t] = jnp.concatenate([b, pb, zpad], axis=1)


def _tables_call(xt, pf, wi, wj, b1pad, interpret=False):
    return pl.pallas_call(
        _tables_body,
        grid=(N // BN,),
        in_specs=[
            pl.BlockSpec((T, BN, F), lambda i: (0, i, 0)),
            pl.BlockSpec((T, BN, 8), lambda i: (0, i, 0)),
            pl.BlockSpec((F, HID), lambda i: (0, 0)),
            pl.BlockSpec((F, HID), lambda i: (0, 0)),
            pl.BlockSpec((8, HID), lambda i: (0, 0)),
        ],
        out_specs=[
            pl.BlockSpec((T, BN, F), lambda i: (0, i, 0)),
            pl.BlockSpec((T, BN, F), lambda i: (0, i, 0)),
        ],
        out_shape=[
            jax.ShapeDtypeStruct((T, N, F), jnp.float32),
            jax.ShapeDtypeStruct((T, N, F), jnp.float32),
        ],
        interpret=interpret,
    )(xt, pf, wi, wj, b1pad)


def _edge_body(gd_ref, gs_ref, ea_ref, wea_ref, w2_ref, wp1_ref, ec_ref,
               mp_ref):
    ec = ec_ref[...]
    wd = ec[0:1, :HID]
    b2 = ec[1:2, :HID]
    bp1 = ec[2:3, :HID]
    bp2 = ec[3:4, 0:1]
    wp2 = ec[4:5, :HID]
    gd = gd_ref[...]
    gs = gs_ref[...]
    d8 = gd[:, HID:HID + 8] - gs[:, HID:HID + 8]
    d2 = jnp.sum(d8 * d8, axis=1, keepdims=True)
    h0 = gd[:, :HID] + gs[:, :HID] + d2 * wd + _dot(ea_ref[...], wea_ref[...])
    h1 = _silu(h0)
    m = _silu(_dot(h1, w2_ref[...]) + b2)
    q = _silu(_dot(m, wp1_ref[...]) + bp1)
    coef = jnp.sum(q * wp2, axis=1, keepdims=True) + bp2
    zpad = jnp.zeros((BE, F - HID - 8), jnp.float32)
    mp_ref[...] = jnp.concatenate([m, d8 * coef, zpad], axis=1)


def _edge_call(gd, gs, ea, wea, w2, wp1, ec, interpret=False):
    return pl.pallas_call(
        _edge_body,
        grid=(E // BE,),
        in_specs=[
            pl.BlockSpec((BE, F), lambda i: (i, 0)),
            pl.BlockSpec((BE, F), lambda i: (i, 0)),
            pl.BlockSpec((BE, DE), lambda i: (i, 0)),
            pl.BlockSpec((DE, HID), lambda i: (0, 0)),
            pl.BlockSpec((HID, HID), lambda i: (0, 0)),
            pl.BlockSpec((HID, HID), lambda i: (0, 0)),
            pl.BlockSpec((8, 128), lambda i: (0, 0)),
        ],
        out_specs=pl.BlockSpec((BE, F), lambda i: (i, 0)),
        out_shape=jax.ShapeDtypeStruct((E, F), jnp.float32),
        interpret=interpret,
    )(gd, gs, ea, wea, w2, wp1, ec)


def _node_body(xt_ref, om0_ref, om1_ref, pf_ref, vf_ref,
               wv1_ref, wu1x_ref, wu1m_ref, wu2_ref, u_ref, v_ref, nc_ref,
               xo_ref, po_ref, vo_ref):
    ncs = nc_ref[...]
    bv1 = ncs[0:1, :HID]
    bv2 = ncs[1:2, 0:1]
    bu1 = ncs[2:3, :HID]
    bu2 = ncs[3:4, :F]
    wv2 = ncs[4:5, :HID]
    om = (om0_ref, om1_ref)
    xm = []
    for t in range(T):
        x_t = xt_ref[t]
        agg = om[t][0] + om[t][1]
        magg = agg[:, :HID]
        pagg = agg[:, HID:HID + 8]
        hv = _silu(_dot(x_t, wv1_ref[...]) + bv1)
        velc = jnp.sum(hv * wv2, axis=1, keepdims=True) + bv2
        v_new = velc * vf_ref[t] + pagg
        po_ref[t] = pf_ref[t] + v_new
        vo_ref[t] = v_new
        hu = _silu(_dot(x_t, wu1x_ref[...]) + _dot(magg, wu1m_ref[...]) + bu1)
        xm.append(x_t + _dot(hu, wu2_ref[...]) + bu2)
    u_w = u_ref[...]
    v_w = v_ref[...]
    xo_ref[0] = xm[0] + _dot(xm[0], u_w) + _dot(xm[1], v_w)
    xo_ref[1] = xm[1] + _dot(xm[1], u_w) + _dot(xm[0], v_w)


def _node_call(xt, om0, om1, pf, vf, wv1, wu1x, wu1m, wu2, u_w, v_w,
               ncs, interpret=False):
    return pl.pallas_call(
        _node_body,
        grid=(N // BN,),
        in_specs=[
            pl.BlockSpec((T, BN, F), lambda i: (0, i, 0)),
            pl.BlockSpec((NC, BN, F), lambda i: (0, i, 0)),
            pl.BlockSpec((NC, BN, F), lambda i: (0, i, 0)),
            pl.BlockSpec((T, BN, 8), lambda i: (0, i, 0)),
            pl.BlockSpec((T, BN, 8), lambda i: (0, i, 0)),
            pl.BlockSpec((F, HID), lambda i: (0, 0)),
            pl.BlockSpec((F, HID), lambda i: (0, 0)),
            pl.BlockSpec((HID, HID), lambda i: (0, 0)),
            pl.BlockSpec((HID, F), lambda i: (0, 0)),
            pl.BlockSpec((F, F), lambda i: (0, 0)),
            pl.BlockSpec((F, F), lambda i: (0, 0)),
            pl.BlockSpec((8, 128), lambda i: (0, 0)),
        ],
        out_specs=[
            pl.BlockSpec((T, BN, F), lambda i: (0, i, 0)),
            pl.BlockSpec((T, BN, 8), lambda i: (0, i, 0)),
            pl.BlockSpec((T, BN, 8), lambda i: (0, i, 0)),
        ],
        out_shape=[
            jax.ShapeDtypeStruct((T, N, F), jnp.float32),
            jax.ShapeDtypeStruct((T, N, 8), jnp.float32),
            jax.ShapeDtypeStruct((T, N, 8), jnp.float32),
        ],
        interpret=interpret,
    )(xt, om0, om1, pf, vf, wv1, wu1x, wu1m, wu2, u_w, v_w, ncs)


def _sc_mesh():
    return plsc.VectorSubcoreMesh(
        core_axis_name="c", subcore_axis_name="s", num_cores=NC,
        num_subcores=NS)


def _gather_call(tabd, tabs, dst, src):
    @functools.partial(
        pl.kernel,
        out_type=[
            jax.ShapeDtypeStruct((E, F), jnp.float32),
            jax.ShapeDtypeStruct((E, F), jnp.float32),
        ],
        mesh=_sc_mesh(),
        scratch_types=[
            pltpu.VMEM((2, CH), jnp.int32),
            pltpu.VMEM((2, CH, F), jnp.float32),
            pltpu.VMEM_SHARED((N, F), jnp.float32),
            pltpu.SemaphoreType.DMA,
            pltpu.SemaphoreType.DMA,
            pltpu.SemaphoreType.DMA,
        ],
    )
    def k(tabd_h, tabs_h, dst_h, src_h, gd_h, gs_h,
          idx, buf, stab, semi, semg, semw):
        c = lax.axis_index("c")
        s = lax.axis_index("s")

        def stage(tab_h):
            @pl.when(s < NS - 1)
            def _():
                pltpu.sync_copy(tab_h.at[pl.ds(s * STR_A, STR_A)],
                                stab.at[pl.ds(s * STR_A, STR_A)])

            @pl.when(s == NS - 1)
            def _():
                pltpu.sync_copy(tab_h.at[pl.ds((NS - 1) * STR_A, STR_B)],
                                stab.at[pl.ds((NS - 1) * STR_A, STR_B)])

        @pl.when(c == 0)
        def _():
            stage(tabd_h)

        @pl.when(c == 1)
        def _():
            stage(tabs_h)

        plsc.subcore_barrier()
        nk = (NCHUNK + NS - 1 - s) // NS

        def pipeline(src_idx_h, out_h):
            def chunk_base(i):
                return (s + i * NS) * CH

            def fire_idx(i, slot):
                pltpu.async_copy(src_idx_h.at[pl.ds(chunk_base(i), CH)],
                                 idx.at[slot], semi)

            def wait_idx(slot):
                pltpu.make_async_copy(src_idx_h.at[pl.ds(0, CH)],
                                      idx.at[slot], semi).wait()

            def fire_gather(slot):
                pltpu.async_copy(stab.at[idx.at[slot]], buf.at[slot], semg)

            def wait_gather(slot):
                pltpu.make_async_copy(stab.at[idx.at[slot]], buf.at[slot],
                                      semg).wait()

            def fire_wb(i, slot):
                pltpu.async_copy(buf.at[slot],
                                 out_h.at[pl.ds(chunk_base(i), CH)], semw)

            def wait_wb(slot):
                pltpu.make_async_copy(buf.at[slot], out_h.at[pl.ds(0, CH)],
                                      semw).wait()

            fire_idx(0, 0)

            def body(i, carry):
                slot = lax.rem(i, 2)
                wait_idx(slot)

                @pl.when(i >= 2)
                def _():
                    wait_wb(slot)

                fire_gather(slot)

                @pl.when(i >= 1)
                def _():
                    wait_gather(1 - slot)
                    fire_wb(i - 1, 1 - slot)

                @pl.when(i + 1 < nk)
                def _():
                    fire_idx(i + 1, 1 - slot)

                return carry

            lax.fori_loop(0, nk, body, 0)
            last = lax.rem(nk - 1, 2)
            wait_gather(last)
            fire_wb(nk - 1, last)

            @pl.when(nk >= 2)
            def _():
                wait_wb(1 - last)

            wait_wb(last)

        @pl.when(c == 0)
        def _():
            pipeline(dst_h, gd_h)

        @pl.when(c == 1)
        def _():
            pipeline(src_h, gs_h)

    return k(tabd, tabs, dst, src)


def _scatter_call(mp, dst, zm):
    @functools.partial(
        pl.kernel,
        out_type=jax.ShapeDtypeStruct((NC * N, F), jnp.float32),
        mesh=_sc_mesh(),
        scratch_types=[
            pltpu.VMEM((2, CH), jnp.int32),
            pltpu.VMEM((2, CH, F), jnp.float32),
            pltpu.VMEM_SHARED((N, F), jnp.float32),
            pltpu.SemaphoreType.DMA,
            pltpu.SemaphoreType.DMA,
            pltpu.SemaphoreType.DMA,
        ],
    )
    def k(mp_h, dst_h, zm_h, om_h, idx2, bufm, acc, semi, semm, sema):
        c = lax.axis_index("c")
        s = lax.axis_index("s")
        wid = s * NC + c

        @pl.when(s < NS - 1)
        def _():
            pltpu.sync_copy(zm_h.at[pl.ds(0, STR_A)],
                            acc.at[pl.ds(s * STR_A, STR_A)])

        @pl.when(s == NS - 1)
        def _():
            pltpu.sync_copy(zm_h, acc.at[pl.ds((NS - 1) * STR_A, STR_B)])

        plsc.subcore_barrier()
        nk = (NCHUNK + NW - 1 - wid) // NW

        def chunk_base(i):
            return (wid + i * NW) * CH

        def fire_in(i, slot):
            base = chunk_base(i)
            pltpu.async_copy(dst_h.at[pl.ds(base, CH)], idx2.at[slot], semi)
            pltpu.async_copy(mp_h.at[pl.ds(base, CH)], bufm.at[slot], semm)

        def wait_in(slot):
            pltpu.make_async_copy(dst_h.at[pl.ds(0, CH)], idx2.at[slot],
                                  semi).wait()
            pltpu.make_async_copy(mp_h.at[pl.ds(0, CH)], bufm.at[slot],
                                  semm).wait()

        def fire_add(slot):
            pltpu.async_copy(bufm.at[slot], acc.at[idx2.at[slot]], sema,
                             add=True)

        def wait_add(slot):
            pltpu.make_async_copy(bufm.at[slot], acc.at[idx2.at[slot]],
                                  sema).wait()

        fire_in(0, 0)

        def body(i, carry):
            slot = lax.rem(i, 2)
            wait_in(slot)
            fire_add(slot)

            @pl.when(i >= 1)
            def _():
                wait_add(1 - slot)

            @pl.when(i + 1 < nk)
            def _():
                fire_in(i + 1, 1 - slot)

            return carry

        lax.fori_loop(0, nk, body, 0)
        wait_add(lax.rem(nk - 1, 2))
        plsc.subcore_barrier()

        @pl.when(s < NS - 1)
        def _():
            pltpu.sync_copy(acc.at[pl.ds(s * STR_A, STR_A)],
                            om_h.at[pl.ds(c * N + s * STR_A, STR_A)])

        @pl.when(s == NS - 1)
        def _():
            pltpu.sync_copy(acc.at[pl.ds((NS - 1) * STR_A, STR_B)],
                            om_h.at[pl.ds(c * N + (NS - 1) * STR_A, STR_B)])

    return k(mp, dst, zm)


def _pad_rows(v, rows=8):
    v2 = v.reshape(1, -1)
    return jnp.concatenate(
        [v2, jnp.zeros((rows - 1, v2.shape[1]), jnp.float32)], axis=0)


def _pack_consts(rows, width=128):
    out = []
    for r in rows:
        r = jnp.asarray(r, jnp.float32).reshape(-1)
        out.append(jnp.concatenate(
            [r, jnp.zeros((width - r.shape[0],), jnp.float32)]))
    while len(out) < 8:
        out.append(jnp.zeros((width,), jnp.float32))
    return jnp.stack(out)


def kernel(x, pos, vel, edge_attr, params, edge_index):
    src = edge_index[0].astype(jnp.int32)
    dst = edge_index[1].astype(jnp.int32)

    half = TED // 2
    logs = np.log(MAX_T) / (half - 1)
    freqs = np.exp(-np.arange(half) * logs)
    args = np.arange(T)[:, None] * freqs[None, :]
    emb = jnp.asarray(
        np.concatenate([np.sin(args), np.cos(args)], axis=-1), jnp.float32)
    c = emb @ params['lin_w'][F:] + params['lin_b']
    cpad = jnp.concatenate([c, jnp.zeros((8 - T, F), jnp.float32)], axis=0)

    xt = _init_call(x, params['lin_w'][:F], cpad)

    pf = jnp.concatenate([pos, jnp.zeros((N, 8 - P), jnp.float32)], axis=1)
    pf = jnp.broadcast_to(pf[None], (T, N, 8))
    vf = jnp.concatenate([vel, jnp.zeros((N, 8 - P), jnp.float32)], axis=1)
    vf = jnp.broadcast_to(vf[None], (T, N, 8))

    zm = jnp.zeros((STR_B, F), jnp.float32)

    for lp in params['layers']:
        (w1, b1), (w2, b2) = lp['msg']
        w_xi, w_xj, wd, wea = w1[:F], w1[F:2 * F], w1[2 * F], w1[2 * F + 1:]
        (wp1, bp1), (wp2, bp2) = lp['posm']
        (wv1, bv1), (wv2, bv2) = lp['velm']
        (wu1, bu1), (wu2, bu2) = lp['upd']
        wu1x, wu1m = wu1[:F], wu1[F:]
        u_w = (lp['wr'][0] + lp['wr'][1]) * 0.5
        v_w = (lp['wr'][0] - lp['wr'][1]) * 0.5

        tabd, tabs = _tables_call(xt, pf, w_xi, w_xj, _pad_rows(b1))
        ec = _pack_consts([wd, b2, bp1, bp2, wp2.reshape(-1)])
        ncs = _pack_consts([bv1, bv2, bu1, bu2, wv2.reshape(-1)])

        oms = []
        for t in range(T):
            gd, gs = _gather_call(tabd[t], tabs[t], dst, src)
            mp = _edge_call(gd, gs, edge_attr, wea, w2, wp1, ec)
            om = _scatter_call(mp, dst, zm)
            oms.append(om.reshape(NC, N, F))

        xt, pf, vf = _node_call(
            xt, oms[0], oms[1], pf, vf,
            wv1, wu1x, wu1m, wu2, u_w, v_w, ncs)

    return xt, pf[..., :P], vf[..., :P]


# speedup vs baseline: 11.5418x; 1.0028x over previous
import functools

import numpy as np
import jax
import jax.numpy as jnp
from jax import lax
from jax.experimental import pallas as pl
from jax.experimental.pallas import tpu as pltpu
from jax.experimental.pallas import tpu_sc as plsc

T = 2
N = 10000
E = 320000
F = 128
DE = 16
P = 3
HID = 64
TED = 16
MAX_T = 10000

NC = 2
NS = 16
NW = NC * NS
CH = 128
NCHUNK = E // CH
STR_A = 624
STR_B = N - (NS - 1) * STR_A

BN = 2000
BE = 4000


def _silu(x):
    return x * (1.0 / (1.0 + jnp.exp(-x)))


def _dot(a, b):
    return jnp.dot(a, b, preferred_element_type=jnp.float32)


def _init_body(x_ref, wx_ref, c_ref, out_ref):
    y = _dot(x_ref[...], wx_ref[...])
    out_ref[...] = y[None] + c_ref[0:T][:, None, :]


def _init_call(x, wx, cpad, interpret=False):
    return pl.pallas_call(
        _init_body,
        grid=(N // BN,),
        in_specs=[
            pl.BlockSpec((BN, F), lambda i: (i, 0)),
            pl.BlockSpec((F, F), lambda i: (0, 0)),
            pl.BlockSpec((8, F), lambda i: (0, 0)),
        ],
        out_specs=pl.BlockSpec((T, BN, F), lambda i: (0, i, 0)),
        out_shape=jax.ShapeDtypeStruct((T, N, F), jnp.float32),
        interpret=interpret,
    )(x, wx, cpad)


def _tables_body(xt_ref, pf_ref, wi_ref, wj_ref, b1_ref, td_ref, ts_ref):
    zpad = jnp.zeros((BN, F - HID - 8), jnp.float32)
    for t in range(T):
        xb = xt_ref[t]
        a = _dot(xb, wi_ref[...]) + b1_ref[0:1]
        b = _dot(xb, wj_ref[...])
        pb = pf_ref[t]
        td_ref[t] = jnp.concatenate([a, pb, zpad], axis=1)
        ts_ref[t] = jnp.concatenate([b, pb, zpad], axis=1)


def _tables_call(xt, pf, wi, wj, b1pad, interpret=False):
    return pl.pallas_call(
        _tables_body,
        grid=(N // BN,),
        in_specs=[
            pl.BlockSpec((T, BN, F), lambda i: (0, i, 0)),
            pl.BlockSpec((T, BN, 8), lambda i: (0, i, 0)),
            pl.BlockSpec((F, HID), lambda i: (0, 0)),
            pl.BlockSpec((F, HID), lambda i: (0, 0)),
            pl.BlockSpec((8, HID), lambda i: (0, 0)),
        ],
        out_specs=[
            pl.BlockSpec((T, BN, F), lambda i: (0, i, 0)),
            pl.BlockSpec((T, BN, F), lambda i: (0, i, 0)),
        ],
        out_shape=[
            jax.ShapeDtypeStruct((T, N, F), jnp.float32),
            jax.ShapeDtypeStruct((T, N, F), jnp.float32),
        ],
        interpret=interpret,
    )(xt, pf, wi, wj, b1pad)


def _edge_body(gd_ref, gs_ref, ea_ref, wea_ref, w2_ref, wp1_ref, ec_ref,
               mp_ref):
    ec = ec_ref[...]
    wd = ec[0:1, :HID]
    b2 = ec[1:2, :HID]
    bp1 = ec[2:3, :HID]
    bp2 = ec[3:4, 0:1]
    wp2 = ec[4:5, :HID]
    gd = gd_ref[...]
    gs = gs_ref[...]
    d8 = gd[:, HID:HID + 8] - gs[:, HID:HID + 8]
    d2 = jnp.sum(d8 * d8, axis=1, keepdims=True)
    h0 = gd[:, :HID] + gs[:, :HID] + d2 * wd + _dot(ea_ref[...], wea_ref[...])
    h1 = _silu(h0)
    m = _silu(_dot(h1, w2_ref[...]) + b2)
    q = _silu(_dot(m, wp1_ref[...]) + bp1)
    coef = jnp.sum(q * wp2, axis=1, keepdims=True) + bp2
    zpad = jnp.zeros((BE, F - HID - 8), jnp.float32)
    mp_ref[...] = jnp.concatenate([m, d8 * coef, zpad], axis=1)


def _edge_call(gd, gs, ea, wea, w2, wp1, ec, interpret=False):
    return pl.pallas_call(
        _edge_body,
        grid=(E // BE,),
        in_specs=[
            pl.BlockSpec((BE, F), lambda i: (i, 0)),
            pl.BlockSpec((BE, F), lambda i: (i, 0)),
            pl.BlockSpec((BE, DE), lambda i: (i, 0)),
            pl.BlockSpec((DE, HID), lambda i: (0, 0)),
            pl.BlockSpec((HID, HID), lambda i: (0, 0)),
            pl.BlockSpec((HID, HID), lambda i: (0, 0)),
            pl.BlockSpec((8, 128), lambda i: (0, 0)),
        ],
        out_specs=pl.BlockSpec((BE, F), lambda i: (i, 0)),
        out_shape=jax.ShapeDtypeStruct((E, F), jnp.float32),
        interpret=interpret,
    )(gd, gs, ea, wea, w2, wp1, ec)


def _node_body(xt_ref, om0_ref, om1_ref, pf_ref, vf_ref,
               wv1_ref, wu1x_ref, wu1m_ref, wu2_ref, u_ref, v_ref, nc_ref,
               xo_ref, po_ref, vo_ref):
    ncs = nc_ref[...]
    bv1 = ncs[0:1, :HID]
    bv2 = ncs[1:2, 0:1]
    bu1 = ncs[2:3, :HID]
    bu2 = ncs[3:4, :F]
    wv2 = ncs[4:5, :HID]
    om = (om0_ref, om1_ref)
    xm = []
    for t in range(T):
        x_t = xt_ref[t]
        agg = om[t][0] + om[t][1]
        magg = agg[:, :HID]
        pagg = agg[:, HID:HID + 8]
        hv = _silu(_dot(x_t, wv1_ref[...]) + bv1)
        velc = jnp.sum(hv * wv2, axis=1, keepdims=True) + bv2
        v_new = velc * vf_ref[t] + pagg
        po_ref[t] = pf_ref[t] + v_new
        vo_ref[t] = v_new
        hu = _silu(_dot(x_t, wu1x_ref[...]) + _dot(magg, wu1m_ref[...]) + bu1)
        xm.append(x_t + _dot(hu, wu2_ref[...]) + bu2)
    u_w = u_ref[...]
    v_w = v_ref[...]
    xo_ref[0] = xm[0] + _dot(xm[0], u_w) + _dot(xm[1], v_w)
    xo_ref[1] = xm[1] + _dot(xm[1], u_w) + _dot(xm[0], v_w)


def _node_call(xt, om0, om1, pf, vf, wv1, wu1x, wu1m, wu2, u_w, v_w,
               ncs, interpret=False):
    return pl.pallas_call(
        _node_body,
        grid=(N // BN,),
        in_specs=[
            pl.BlockSpec((T, BN, F), lambda i: (0, i, 0)),
            pl.BlockSpec((NC, BN, F), lambda i: (0, i, 0)),
            pl.BlockSpec((NC, BN, F), lambda i: (0, i, 0)),
            pl.BlockSpec((T, BN, 8), lambda i: (0, i, 0)),
            pl.BlockSpec((T, BN, 8), lambda i: (0, i, 0)),
            pl.BlockSpec((F, HID), lambda i: (0, 0)),
            pl.BlockSpec((F, HID), lambda i: (0, 0)),
            pl.BlockSpec((HID, HID), lambda i: (0, 0)),
            pl.BlockSpec((HID, F), lambda i: (0, 0)),
            pl.BlockSpec((F, F), lambda i: (0, 0)),
            pl.BlockSpec((F, F), lambda i: (0, 0)),
            pl.BlockSpec((8, 128), lambda i: (0, 0)),
        ],
        out_specs=[
            pl.BlockSpec((T, BN, F), lambda i: (0, i, 0)),
            pl.BlockSpec((T, BN, 8), lambda i: (0, i, 0)),
            pl.BlockSpec((T, BN, 8), lambda i: (0, i, 0)),
        ],
        out_shape=[
            jax.ShapeDtypeStruct((T, N, F), jnp.float32),
            jax.ShapeDtypeStruct((T, N, 8), jnp.float32),
            jax.ShapeDtypeStruct((T, N, 8), jnp.float32),
        ],
        interpret=interpret,
    )(xt, om0, om1, pf, vf, wv1, wu1x, wu1m, wu2, u_w, v_w, ncs)


def _sc_mesh():
    return plsc.VectorSubcoreMesh(
        core_axis_name="c", subcore_axis_name="s", num_cores=NC,
        num_subcores=NS)


def _gather_call(tabd, tabs, dst, src):
    @functools.partial(
        pl.kernel,
        out_type=[
            jax.ShapeDtypeStruct((E, F), jnp.float32),
            jax.ShapeDtypeStruct((E, F), jnp.float32),
        ],
        cost_estimate=pl.CostEstimate(
            flops=0, bytes_accessed=4 * E * F * 4, transcendentals=0),
        mesh=_sc_mesh(),
        scratch_types=[
            pltpu.VMEM((2, CH), jnp.int32),
            pltpu.VMEM((2, CH, F), jnp.float32),
            pltpu.VMEM_SHARED((N, F), jnp.float32),
            pltpu.SemaphoreType.DMA,
            pltpu.SemaphoreType.DMA,
            pltpu.SemaphoreType.DMA,
        ],
    )
    def k(tabd_h, tabs_h, dst_h, src_h, gd_h, gs_h,
          idx, buf, stab, semi, semg, semw):
        c = lax.axis_index("c")
        s = lax.axis_index("s")

        def stage(tab_h):
            @pl.when(s < NS - 1)
            def _():
                pltpu.sync_copy(tab_h.at[pl.ds(s * STR_A, STR_A)],
                                stab.at[pl.ds(s * STR_A, STR_A)])

            @pl.when(s == NS - 1)
            def _():
                pltpu.sync_copy(tab_h.at[pl.ds((NS - 1) * STR_A, STR_B)],
                                stab.at[pl.ds((NS - 1) * STR_A, STR_B)])

        @pl.when(c == 0)
        def _():
            stage(tabd_h)

        @pl.when(c == 1)
        def _():
            stage(tabs_h)

        plsc.subcore_barrier()
        nk = (NCHUNK + NS - 1 - s) // NS

        def pipeline(src_idx_h, out_h):
            def chunk_base(i):
                return (s + i * NS) * CH

            def fire_idx(i, slot):
                pltpu.async_copy(src_idx_h.at[pl.ds(chunk_base(i), CH)],
                                 idx.at[slot], semi)

            def wait_idx(slot):
                pltpu.make_async_copy(src_idx_h.at[pl.ds(0, CH)],
                                      idx.at[slot], semi).wait()

            def fire_gather(slot):
                pltpu.async_copy(stab.at[idx.at[slot]], buf.at[slot], semg)

            def wait_gather(slot):
                pltpu.make_async_copy(stab.at[idx.at[slot]], buf.at[slot],
                                      semg).wait()

            def fire_wb(i, slot):
                pltpu.async_copy(buf.at[slot],
                                 out_h.at[pl.ds(chunk_base(i), CH)], semw)

            def wait_wb(slot):
                pltpu.make_async_copy(buf.at[slot], out_h.at[pl.ds(0, CH)],
                                      semw).wait()

            fire_idx(0, 0)

            def body(i, carry):
                slot = lax.rem(i, 2)
                wait_idx(slot)

                @pl.when(i >= 2)
                def _():
                    wait_wb(slot)

                fire_gather(slot)

                @pl.when(i >= 1)
                def _():
                    wait_gather(1 - slot)
                    fire_wb(i - 1, 1 - slot)

                @pl.when(i + 1 < nk)
                def _():
                    fire_idx(i + 1, 1 - slot)

                return carry

            lax.fori_loop(0, nk, body, 0)
            last = lax.rem(nk - 1, 2)
            wait_gather(last)
            fire_wb(nk - 1, last)

            @pl.when(nk >= 2)
            def _():
                wait_wb(1 - last)

            wait_wb(last)

        @pl.when(c == 0)
        def _():
            pipeline(dst_h, gd_h)

        @pl.when(c == 1)
        def _():
            pipeline(src_h, gs_h)

    return k(tabd, tabs, dst, src)


def _scatter_call(mp, dst, zm):
    @functools.partial(
        pl.kernel,
        out_type=jax.ShapeDtypeStruct((NC * N, F), jnp.float32),
        cost_estimate=pl.CostEstimate(
            flops=0, bytes_accessed=2 * E * F * 4, transcendentals=0),
        mesh=_sc_mesh(),
        scratch_types=[
            pltpu.VMEM((2, CH), jnp.int32),
            pltpu.VMEM((2, CH, F), jnp.float32),
            pltpu.VMEM_SHARED((N, F), jnp.float32),
            pltpu.SemaphoreType.DMA,
            pltpu.SemaphoreType.DMA,
            pltpu.SemaphoreType.DMA,
        ],
    )
    def k(mp_h, dst_h, zm_h, om_h, idx2, bufm, acc, semi, semm, sema):
        c = lax.axis_index("c")
        s = lax.axis_index("s")
        wid = s * NC + c

        @pl.when(s < NS - 1)
        def _():
            pltpu.sync_copy(zm_h.at[pl.ds(0, STR_A)],
                            acc.at[pl.ds(s * STR_A, STR_A)])

        @pl.when(s == NS - 1)
        def _():
            pltpu.sync_copy(zm_h, acc.at[pl.ds((NS - 1) * STR_A, STR_B)])

        plsc.subcore_barrier()
        nk = (NCHUNK + NW - 1 - wid) // NW

        def chunk_base(i):
            return (wid + i * NW) * CH

        def fire_in(i, slot):
            base = chunk_base(i)
            pltpu.async_copy(dst_h.at[pl.ds(base, CH)], idx2.at[slot], semi)
            pltpu.async_copy(mp_h.at[pl.ds(base, CH)], bufm.at[slot], semm)

        def wait_in(slot):
            pltpu.make_async_copy(dst_h.at[pl.ds(0, CH)], idx2.at[slot],
                                  semi).wait()
            pltpu.make_async_copy(mp_h.at[pl.ds(0, CH)], bufm.at[slot],
                                  semm).wait()

        def fire_add(slot):
            pltpu.async_copy(bufm.at[slot], acc.at[idx2.at[slot]], sema,
                             add=True)

        def wait_add(slot):
            pltpu.make_async_copy(bufm.at[slot], acc.at[idx2.at[slot]],
                                  sema).wait()

        fire_in(0, 0)

        def body(i, carry):
            slot = lax.rem(i, 2)
            wait_in(slot)
            fire_add(slot)

            @pl.when(i >= 1)
            def _():
                wait_add(1 - slot)

            @pl.when(i + 1 < nk)
            def _():
                fire_in(i + 1, 1 - slot)

            return carry

        lax.fori_loop(0, nk, body, 0)
        wait_add(lax.rem(nk - 1, 2))
        plsc.subcore_barrier()

        @pl.when(s < NS - 1)
        def _():
            pltpu.sync_copy(acc.at[pl.ds(s * STR_A, STR_A)],
                            om_h.at[pl.ds(c * N + s * STR_A, STR_A)])

        @pl.when(s == NS - 1)
        def _():
            pltpu.sync_copy(acc.at[pl.ds((NS - 1) * STR_A, STR_B)],
                            om_h.at[pl.ds(c * N + (NS - 1) * STR_A, STR_B)])

    return k(mp, dst, zm)


def _pad_rows(v, rows=8):
    v2 = v.reshape(1, -1)
    return jnp.concatenate(
        [v2, jnp.zeros((rows - 1, v2.shape[1]), jnp.float32)], axis=0)


def _pack_consts(rows, width=128):
    out = []
    for r in rows:
        r = jnp.asarray(r, jnp.float32).reshape(-1)
        out.append(jnp.concatenate(
            [r, jnp.zeros((width - r.shape[0],), jnp.float32)]))
    while len(out) < 8:
        out.append(jnp.zeros((width,), jnp.float32))
    return jnp.stack(out)


def kernel(x, pos, vel, edge_attr, params, edge_index):
    src = edge_index[0].astype(jnp.int32)
    dst = edge_index[1].astype(jnp.int32)

    half = TED // 2
    logs = np.log(MAX_T) / (half - 1)
    freqs = np.exp(-np.arange(half) * logs)
    args = np.arange(T)[:, None] * freqs[None, :]
    emb = jnp.asarray(
        np.concatenate([np.sin(args), np.cos(args)], axis=-1), jnp.float32)
    c = emb @ params['lin_w'][F:] + params['lin_b']
    cpad = jnp.concatenate([c, jnp.zeros((8 - T, F), jnp.float32)], axis=0)

    xt = _init_call(x, params['lin_w'][:F], cpad)

    pf = jnp.concatenate([pos, jnp.zeros((N, 8 - P), jnp.float32)], axis=1)
    pf = jnp.broadcast_to(pf[None], (T, N, 8))
    vf = jnp.concatenate([vel, jnp.zeros((N, 8 - P), jnp.float32)], axis=1)
    vf = jnp.broadcast_to(vf[None], (T, N, 8))

    zm = jnp.zeros((STR_B, F), jnp.float32)

    for lp in params['layers']:
        (w1, b1), (w2, b2) = lp['msg']
        w_xi, w_xj, wd, wea = w1[:F], w1[F:2 * F], w1[2 * F], w1[2 * F + 1:]
        (wp1, bp1), (wp2, bp2) = lp['posm']
        (wv1, bv1), (wv2, bv2) = lp['velm']
        (wu1, bu1), (wu2, bu2) = lp['upd']
        wu1x, wu1m = wu1[:F], wu1[F:]
        u_w = (lp['wr'][0] + lp['wr'][1]) * 0.5
        v_w = (lp['wr'][0] - lp['wr'][1]) * 0.5

        tabd, tabs = _tables_call(xt, pf, w_xi, w_xj, _pad_rows(b1))
        ec = _pack_consts([wd, b2, bp1, bp2, wp2.reshape(-1)])
        ncs = _pack_consts([bv1, bv2, bu1, bu2, wv2.reshape(-1)])

        oms = []
        for t in range(T):
            gd, gs = _gather_call(tabd[t], tabs[t], dst, src)
            mp = _edge_call(gd, gs, edge_attr, wea, w2, wp1, ec)
            om = _scatter_call(mp, dst, zm)
            oms.append(om.reshape(NC, N, F))

        xt, pf, vf = _node_call(
            xt, oms[0], oms[1], pf, vf,
            wv1, wu1x, wu1m, wu2, u_w, v_w, ncs)

    return xt, pf[..., :P], vf[..., :P]


# speedup vs baseline: 11.5424x; 1.0001x over previous
import functools

import numpy as np
import jax
import jax.numpy as jnp
from jax import lax
from jax.experimental import pallas as pl
from jax.experimental.pallas import tpu as pltpu
from jax.experimental.pallas import tpu_sc as plsc

T = 2
N = 10000
E = 320000
F = 128
DE = 16
P = 3
HID = 64
TED = 16
MAX_T = 10000

NC = 2
NS = 16
NW = NC * NS
CH = 128
NCHUNK = E // CH
STR_A = 624
STR_B = N - (NS - 1) * STR_A

BN = 2000
BE = 4000


def _silu(x):
    return x * (1.0 / (1.0 + jnp.exp(-x)))


def _dot(a, b):
    return jnp.dot(a, b, preferred_element_type=jnp.float32)


def _init_body(x_ref, wx_ref, c_ref, out_ref):
    y = _dot(x_ref[...], wx_ref[...])
    out_ref[...] = y[None] + c_ref[0:T][:, None, :]


def _init_call(x, wx, cpad, interpret=False):
    return pl.pallas_call(
        _init_body,
        grid=(N // BN,),
        in_specs=[
            pl.BlockSpec((BN, F), lambda i: (i, 0)),
            pl.BlockSpec((F, F), lambda i: (0, 0)),
            pl.BlockSpec((8, F), lambda i: (0, 0)),
        ],
        out_specs=pl.BlockSpec((T, BN, F), lambda i: (0, i, 0)),
        out_shape=jax.ShapeDtypeStruct((T, N, F), jnp.float32),
        interpret=interpret,
    )(x, wx, cpad)


def _tables_body(xt_ref, pf_ref, wi_ref, wj_ref, b1_ref, td_ref, ts_ref):
    zpad = jnp.zeros((BN, F - HID - 8), jnp.float32)
    for t in range(T):
        xb = xt_ref[t]
        a = _dot(xb, wi_ref[...]) + b1_ref[0:1]
        b = _dot(xb, wj_ref[...])
        pb = pf_ref[t]
        td_ref[t] = jnp.concatenate([a, pb, zpad], axis=1)
        ts_ref[t] = jnp.concatenate([b, pb, zpad], axis=1)


def _tables_call(xt, pf, wi, wj, b1pad, interpret=False):
    return pl.pallas_call(
        _tables_body,
        grid=(N // BN,),
        in_specs=[
            pl.BlockSpec((T, BN, F), lambda i: (0, i, 0)),
            pl.BlockSpec((T, BN, 8), lambda i: (0, i, 0)),
            pl.BlockSpec((F, HID), lambda i: (0, 0)),
            pl.BlockSpec((F, HID), lambda i: (0, 0)),
            pl.BlockSpec((8, HID), lambda i: (0, 0)),
        ],
        out_specs=[
            pl.BlockSpec((T, BN, F), lambda i: (0, i, 0)),
            pl.BlockSpec((T, BN, F), lambda i: (0, i, 0)),
        ],
        out_shape=[
            jax.ShapeDtypeStruct((T, N, F), jnp.float32),
            jax.ShapeDtypeStruct((T, N, F), jnp.float32),
        ],
        interpret=interpret,
    )(xt, pf, wi, wj, b1pad)


def _edge_body(gd_ref, gs_ref, ea_ref, wea_ref, w2_ref, wp1_ref, ec_ref,
               mp_ref):
    ec = ec_ref[...]
    wd = ec[0:1, :HID]
    b2 = ec[1:2, :HID]
    bp1 = ec[2:3, :HID]
    bp2 = ec[3:4, 0:1]
    wp2 = ec[4:5, :HID]
    gd = gd_ref[...]
    gs = gs_ref[...]
    d8 = gd[:, HID:HID + 8] - gs[:, HID:HID + 8]
    d2 = jnp.sum(d8 * d8, axis=1, keepdims=True)
    h0 = gd[:, :HID] + gs[:, :HID] + d2 * wd + _dot(ea_ref[...], wea_ref[...])
    h1 = _silu(h0)
    m = _silu(_dot(h1, w2_ref[...]) + b2)
    q = _silu(_dot(m, wp1_ref[...]) + bp1)
    coef = jnp.sum(q * wp2, axis=1, keepdims=True) + bp2
    zpad = jnp.zeros((BE, F - HID - 8), jnp.float32)
    mp_ref[...] = jnp.concatenate([m, d8 * coef, zpad], axis=1)


def _edge_call(gd, gs, ea, wea, w2, wp1, ec, interpret=False):
    return pl.pallas_call(
        _edge_body,
        grid=(E // BE,),
        in_specs=[
            pl.BlockSpec((BE, F), lambda i: (i, 0)),
            pl.BlockSpec((BE, F), lambda i: (i, 0)),
            pl.BlockSpec((BE, DE), lambda i: (i, 0)),
            pl.BlockSpec((DE, HID), lambda i: (0, 0)),
            pl.BlockSpec((HID, HID), lambda i: (0, 0)),
            pl.BlockSpec((HID, HID), lambda i: (0, 0)),
            pl.BlockSpec((8, 128), lambda i: (0, 0)),
        ],
        out_specs=pl.BlockSpec((BE, F), lambda i: (i, 0)),
        out_shape=jax.ShapeDtypeStruct((E, F), jnp.float32),
        interpret=interpret,
    )(gd, gs, ea, wea, w2, wp1, ec)


def _node_body(xt_ref, om0_ref, om1_ref, pf_ref, vf_ref,
               wv1_ref, wu1x_ref, wu1m_ref, wu2_ref, u_ref, v_ref, nc_ref,
               xo_ref, po_ref, vo_ref):
    ncs = nc_ref[...]
    bv1 = ncs[0:1, :HID]
    bv2 = ncs[1:2, 0:1]
    bu1 = ncs[2:3, :HID]
    bu2 = ncs[3:4, :F]
    wv2 = ncs[4:5, :HID]
    om = (om0_ref, om1_ref)
    xm = []
    for t in range(T):
        x_t = xt_ref[t]
        agg = om[t][0] + om[t][1]
        magg = agg[:, :HID]
        pagg = agg[:, HID:HID + 8]
        hv = _silu(_dot(x_t, wv1_ref[...]) + bv1)
        velc = jnp.sum(hv * wv2, axis=1, keepdims=True) + bv2
        v_new = velc * vf_ref[t] + pagg
        po_ref[t] = pf_ref[t] + v_new
        vo_ref[t] = v_new
        hu = _silu(_dot(x_t, wu1x_ref[...]) + _dot(magg, wu1m_ref[...]) + bu1)
        xm.append(x_t + _dot(hu, wu2_ref[...]) + bu2)
    u_w = u_ref[...]
    v_w = v_ref[...]
    xo_ref[0] = xm[0] + _dot(xm[0], u_w) + _dot(xm[1], v_w)
    xo_ref[1] = xm[1] + _dot(xm[1], u_w) + _dot(xm[0], v_w)


def _node_call(xt, om0, om1, pf, vf, wv1, wu1x, wu1m, wu2, u_w, v_w,
               ncs, interpret=False):
    return pl.pallas_call(
        _node_body,
        grid=(N // BN,),
        in_specs=[
            pl.BlockSpec((T, BN, F), lambda i: (0, i, 0)),
            pl.BlockSpec((NC, BN, F), lambda i: (0, i, 0)),
            pl.BlockSpec((NC, BN, F), lambda i: (0, i, 0)),
            pl.BlockSpec((T, BN, 8), lambda i: (0, i, 0)),
            pl.BlockSpec((T, BN, 8), lambda i: (0, i, 0)),
            pl.BlockSpec((F, HID), lambda i: (0, 0)),
            pl.BlockSpec((F, HID), lambda i: (0, 0)),
            pl.BlockSpec((HID, HID), lambda i: (0, 0)),
            pl.BlockSpec((HID, F), lambda i: (0, 0)),
            pl.BlockSpec((F, F), lambda i: (0, 0)),
            pl.BlockSpec((F, F), lambda i: (0, 0)),
            pl.BlockSpec((8, 128), lambda i: (0, 0)),
        ],
        out_specs=[
            pl.BlockSpec((T, BN, F), lambda i: (0, i, 0)),
            pl.BlockSpec((T, BN, 8), lambda i: (0, i, 0)),
            pl.BlockSpec((T, BN, 8), lambda i: (0, i, 0)),
        ],
        out_shape=[
            jax.ShapeDtypeStruct((T, N, F), jnp.float32),
            jax.ShapeDtypeStruct((T, N, 8), jnp.float32),
            jax.ShapeDtypeStruct((T, N, 8), jnp.float32),
        ],
        interpret=interpret,
    )(xt, om0, om1, pf, vf, wv1, wu1x, wu1m, wu2, u_w, v_w, ncs)


def _sc_mesh():
    return plsc.VectorSubcoreMesh(
        core_axis_name="c", subcore_axis_name="s", num_cores=NC,
        num_subcores=NS)


def _gather_call(tabd, tabs, dst, src):
    @functools.partial(
        pl.kernel,
        out_type=[
            jax.ShapeDtypeStruct((E, F), jnp.float32),
            jax.ShapeDtypeStruct((E, F), jnp.float32),
        ],
        cost_estimate=pl.CostEstimate(
            flops=0, bytes_accessed=4 * E * F * 4, transcendentals=0),
        mesh=_sc_mesh(),
        scratch_types=[
            pltpu.VMEM((3, CH), jnp.int32),
            pltpu.VMEM((3, CH, F), jnp.float32),
            pltpu.VMEM_SHARED((N, F), jnp.float32),
            pltpu.SemaphoreType.DMA,
            pltpu.SemaphoreType.DMA,
            pltpu.SemaphoreType.DMA,
        ],
    )
    def k(tabd_h, tabs_h, dst_h, src_h, gd_h, gs_h,
          idx, buf, stab, semi, semg, semw):
        c = lax.axis_index("c")
        s = lax.axis_index("s")

        def stage(tab_h):
            @pl.when(s < NS - 1)
            def _():
                pltpu.sync_copy(tab_h.at[pl.ds(s * STR_A, STR_A)],
                                stab.at[pl.ds(s * STR_A, STR_A)])

            @pl.when(s == NS - 1)
            def _():
                pltpu.sync_copy(tab_h.at[pl.ds((NS - 1) * STR_A, STR_B)],
                                stab.at[pl.ds((NS - 1) * STR_A, STR_B)])

        @pl.when(c == 0)
        def _():
            stage(tabd_h)

        @pl.when(c == 1)
        def _():
            stage(tabs_h)

        plsc.subcore_barrier()
        nk = (NCHUNK + NS - 1 - s) // NS

        def pipeline(src_idx_h, out_h):
            def chunk_base(i):
                return (s + i * NS) * CH

            def fire_idx(i, slot):
                pltpu.async_copy(src_idx_h.at[pl.ds(chunk_base(i), CH)],
                                 idx.at[slot], semi)

            def wait_idx(slot):
                pltpu.make_async_copy(src_idx_h.at[pl.ds(0, CH)],
                                      idx.at[slot], semi).wait()

            def fire_gather(slot):
                pltpu.async_copy(stab.at[idx.at[slot]], buf.at[slot], semg)

            def wait_gather(slot):
                pltpu.make_async_copy(stab.at[idx.at[slot]], buf.at[slot],
                                      semg).wait()

            def fire_wb(i, slot):
                pltpu.async_copy(buf.at[slot],
                                 out_h.at[pl.ds(chunk_base(i), CH)], semw)

            def wait_wb(slot):
                pltpu.make_async_copy(buf.at[slot], out_h.at[pl.ds(0, CH)],
                                      semw).wait()

            fire_idx(0, 0)

            def body(i, carry):
                slot = lax.rem(i, 3)
                wait_idx(slot)

                @pl.when(i >= 3)
                def _():
                    wait_wb(slot)

                fire_gather(slot)

                @pl.when(i >= 1)
                def _():
                    prev = lax.rem(i + 2, 3)
                    wait_gather(prev)
                    fire_wb(i - 1, prev)

                @pl.when(i + 1 < nk)
                def _():
                    fire_idx(i + 1, lax.rem(i + 1, 3))

                return carry

            lax.fori_loop(0, nk, body, 0)
            last = lax.rem(nk - 1, 3)
            wait_gather(last)
            fire_wb(nk - 1, last)

            @pl.when(nk >= 3)
            def _():
                wait_wb(0)

            @pl.when(nk >= 2)
            def _():
                wait_wb(0)

            wait_wb(0)

        @pl.when(c == 0)
        def _():
            pipeline(dst_h, gd_h)

        @pl.when(c == 1)
        def _():
            pipeline(src_h, gs_h)

    return k(tabd, tabs, dst, src)


def _scatter_call(mp, dst, zm):
    @functools.partial(
        pl.kernel,
        out_type=jax.ShapeDtypeStruct((NC * N, F), jnp.float32),
        cost_estimate=pl.CostEstimate(
            flops=0, bytes_accessed=2 * E * F * 4, transcendentals=0),
        mesh=_sc_mesh(),
        scratch_types=[
            pltpu.VMEM((3, CH), jnp.int32),
            pltpu.VMEM((3, CH, F), jnp.float32),
            pltpu.VMEM_SHARED((N, F), jnp.float32),
            pltpu.SemaphoreType.DMA,
            pltpu.SemaphoreType.DMA,
            pltpu.SemaphoreType.DMA,
        ],
    )
    def k(mp_h, dst_h, zm_h, om_h, idx2, bufm, acc, semi, semm, sema):
        c = lax.axis_index("c")
        s = lax.axis_index("s")
        wid = s * NC + c

        @pl.when(s < NS - 1)
        def _():
            pltpu.sync_copy(zm_h.at[pl.ds(0, STR_A)],
                            acc.at[pl.ds(s * STR_A, STR_A)])

        @pl.when(s == NS - 1)
        def _():
            pltpu.sync_copy(zm_h, acc.at[pl.ds((NS - 1) * STR_A, STR_B)])

        plsc.subcore_barrier()
        nk = (NCHUNK + NW - 1 - wid) // NW

        def chunk_base(i):
            return (wid + i * NW) * CH

        def fire_in(i, slot):
            base = chunk_base(i)
            pltpu.async_copy(dst_h.at[pl.ds(base, CH)], idx2.at[slot], semi)
            pltpu.async_copy(mp_h.at[pl.ds(base, CH)], bufm.at[slot], semm)

        def wait_in(slot):
            pltpu.make_async_copy(dst_h.at[pl.ds(0, CH)], idx2.at[slot],
                                  semi).wait()
            pltpu.make_async_copy(mp_h.at[pl.ds(0, CH)], bufm.at[slot],
                                  semm).wait()

        def fire_add(slot):
            pltpu.async_copy(bufm.at[slot], acc.at[idx2.at[slot]], sema,
                             add=True)

        def wait_add(slot):
            pltpu.make_async_copy(bufm.at[slot], acc.at[idx2.at[slot]],
                                  sema).wait()

        fire_in(0, 0)

        def body(i, carry):
            slot = lax.rem(i, 3)
            wait_in(slot)
            fire_add(slot)

            @pl.when(i >= 1)
            def _():
                wait_add(lax.rem(i + 2, 3))

            @pl.when(i + 1 < nk)
            def _():
                fire_in(i + 1, lax.rem(i + 1, 3))

            return carry

        lax.fori_loop(0, nk, body, 0)
        wait_add(lax.rem(nk - 1, 3))
        plsc.subcore_barrier()

        @pl.when(s < NS - 1)
        def _():
            pltpu.sync_copy(acc.at[pl.ds(s * STR_A, STR_A)],
                            om_h.at[pl.ds(c * N + s * STR_A, STR_A)])

        @pl.when(s == NS - 1)
        def _():
            pltpu.sync_copy(acc.at[pl.ds((NS - 1) * STR_A, STR_B)],
                            om_h.at[pl.ds(c * N + (NS - 1) * STR_A, STR_B)])

    return k(mp, dst, zm)


def _pad_rows(v, rows=8):
    v2 = v.reshape(1, -1)
    return jnp.concatenate(
        [v2, jnp.zeros((rows - 1, v2.shape[1]), jnp.float32)], axis=0)


def _pack_consts(rows, width=128):
    out = []
    for r in rows:
        r = jnp.asarray(r, jnp.float32).reshape(-1)
        out.append(jnp.concatenate(
            [r, jnp.zeros((width - r.shape[0],), jnp.float32)]))
    while len(out) < 8:
        out.append(jnp.zeros((width,), jnp.float32))
    return jnp.stack(out)


def kernel(x, pos, vel, edge_attr, params, edge_index):
    src = edge_index[0].astype(jnp.int32)
    dst = edge_index[1].astype(jnp.int32)

    half = TED // 2
    logs = np.log(MAX_T) / (half - 1)
    freqs = np.exp(-np.arange(half) * logs)
    args = np.arange(T)[:, None] * freqs[None, :]
    emb = jnp.asarray(
        np.concatenate([np.sin(args), np.cos(args)], axis=-1), jnp.float32)
    c = emb @ params['lin_w'][F:] + params['lin_b']
    cpad = jnp.concatenate([c, jnp.zeros((8 - T, F), jnp.float32)], axis=0)

    xt = _init_call(x, params['lin_w'][:F], cpad)

    pf = jnp.concatenate([pos, jnp.zeros((N, 8 - P), jnp.float32)], axis=1)
    pf = jnp.broadcast_to(pf[None], (T, N, 8))
    vf = jnp.concatenate([vel, jnp.zeros((N, 8 - P), jnp.float32)], axis=1)
    vf = jnp.broadcast_to(vf[None], (T, N, 8))

    zm = jnp.zeros((STR_B, F), jnp.float32)

    for lp in params['layers']:
        (w1, b1), (w2, b2) = lp['msg']
        w_xi, w_xj, wd, wea = w1[:F], w1[F:2 * F], w1[2 * F], w1[2 * F + 1:]
        (wp1, bp1), (wp2, bp2) = lp['posm']
        (wv1, bv1), (wv2, bv2) = lp['velm']
        (wu1, bu1), (wu2, bu2) = lp['upd']
        wu1x, wu1m = wu1[:F], wu1[F:]
        u_w = (lp['wr'][0] + lp['wr'][1]) * 0.5
        v_w = (lp['wr'][0] - lp['wr'][1]) * 0.5

        tabd, tabs = _tables_call(xt, pf, w_xi, w_xj, _pad_rows(b1))
        ec = _pack_consts([wd, b2, bp1, bp2, wp2.reshape(-1)])
        ncs = _pack_consts([bv1, bv2, bu1, bu2, wv2.reshape(-1)])

        oms = []
        for t in range(T):
            gd, gs = _gather_call(tabd[t], tabs[t], dst, src)
            mp = _edge_call(gd, gs, edge_attr, wea, w2, wp1, ec)
            om = _scatter_call(mp, dst, zm)
            oms.append(om.reshape(NC, N, F))

        xt, pf, vf = _node_call(
            xt, oms[0], oms[1], pf, vf,
            wv1, wu1x, wu1m, wu2, u_w, v_w, ncs)

    return xt, pf[..., :P], vf[..., :P]
